# baseline (speedup 1.0000x reference)
import functools

import jax
import jax.numpy as jnp
from jax import lax
from jax.experimental import pallas as pl
from jax.experimental.pallas import tpu as pltpu
from jax.experimental.pallas import tpu_sc as plsc

_NUM_DIM = 64
_BATCH = 4096
_SEQ = 200
_CA = 128
_CB = _SEQ - _CA

_info = plsc.get_sparse_core_info()
_NC, _NS = _info.num_cores, _info.num_subcores
_NW = _NC * _NS
_SEQ_PER_W = _BATCH // _NW


@functools.partial(
    pl.kernel,
    out_type=jax.ShapeDtypeStruct((_BATCH, _SEQ, _NUM_DIM), jnp.float32),
    mesh=plsc.VectorSubcoreMesh(core_axis_name="c", subcore_axis_name="s"),
    scratch_types=[
        pltpu.VMEM_SHARED((_SEQ, _NUM_DIM), jnp.float32),
        pltpu.VMEM((_CA,), jnp.int32),
        pltpu.VMEM((_CB,), jnp.int32),
        pltpu.VMEM((_SEQ, _NUM_DIM), jnp.float32),
        pltpu.SemaphoreType.DMA,
    ],
    compiler_params=pltpu.CompilerParams(use_tc_tiling_on_sc=False),
)
def _embed_sc(x_hbm, pe_hbm, table_hbm, out_hbm,
              pe_sh, idxa, idxb, buf, sem):
    wid = lax.axis_index("s") * _NC + lax.axis_index("c")
    base = wid * _SEQ_PER_W

    @pl.when(lax.axis_index("s") == 0)
    def _():
        pltpu.sync_copy(pe_hbm, pe_sh)

    plsc.subcore_barrier()

    def step(i, carry):
        s = base + i
        pltpu.sync_copy(x_hbm.at[s, pl.ds(0, _CA)], idxa)
        pltpu.sync_copy(x_hbm.at[s, pl.ds(_CA, _CB)], idxb)
        pltpu.sync_copy(pe_sh, buf)
        pltpu.async_copy(table_hbm.at[idxa], buf.at[pl.ds(0, _CA)],
                         sem, add=True).wait()
        pltpu.async_copy(table_hbm.at[idxb], buf.at[pl.ds(_CA, _CB)],
                         sem, add=True).wait()
        pltpu.sync_copy(buf, out_hbm.at[s])
        return carry

    lax.fori_loop(0, _SEQ_PER_W, step, 0)


def kernel(x, table, pe):
    return _embed_sc(x.astype(jnp.int32), pe[0, :_SEQ], table)


# speedup vs baseline: 1.2664x; 1.2664x over previous
import functools

import jax
import jax.numpy as jnp
from jax import lax
from jax.experimental import pallas as pl
from jax.experimental.pallas import tpu as pltpu
from jax.experimental.pallas import tpu_sc as plsc

_NUM_DIM = 64
_PAD_DIM = 128
_BATCH = 4096
_SEQ = 200

_info = plsc.get_sparse_core_info()
_NC, _NS = _info.num_cores, _info.num_subcores
_NW = _NC * _NS
_TOTAL = _BATCH * _SEQ
_ROWS_PER_W = _TOTAL // _NW
_CH = 128
_NCHUNK = _ROWS_PER_W // _CH
_PE2 = _SEQ + _CH - 8


@functools.partial(
    pl.kernel,
    out_type=jax.ShapeDtypeStruct((_TOTAL, _PAD_DIM), jnp.float32),
    mesh=plsc.VectorSubcoreMesh(core_axis_name="c", subcore_axis_name="s"),
    scratch_types=[
        pltpu.VMEM_SHARED((_PE2, _PAD_DIM), jnp.float32),
        pltpu.VMEM((_CH,), jnp.int32),
        pltpu.VMEM((_CH, _NUM_DIM), jnp.float32),
        pltpu.SemaphoreType.DMA,
    ],
    compiler_params=pltpu.CompilerParams(use_tc_tiling_on_sc=False),
)
def _embed_sc(x_hbm, pe_hbm, table_hbm, out_hbm, pe_sh, idx_v, buf, sem):
    wid = lax.axis_index("s") * _NC + lax.axis_index("c")
    base = wid * _ROWS_PER_W

    @pl.when(lax.axis_index("s") == 0)
    def _():
        pltpu.sync_copy(pe_hbm, pe_sh)

    plsc.subcore_barrier()

    def chunk(c, carry):
        off = base + c * _CH
        r = (c * _CH) % _SEQ
        pltpu.sync_copy(x_hbm.at[pl.ds(off, _CH)], idx_v)
        pltpu.sync_copy(pe_sh.at[pl.ds(r, _CH), pl.ds(0, _NUM_DIM)], buf)
        pltpu.async_copy(table_hbm.at[idx_v], buf, sem, add=True).wait()
        pltpu.sync_copy(buf,
                        out_hbm.at[pl.ds(off, _CH), pl.ds(0, _NUM_DIM)])
        return carry

    lax.fori_loop(0, _NCHUNK, chunk, 0)


def kernel(x, table, pe):
    pe_rows = pe[0, :_SEQ]
    pe2 = jnp.concatenate([pe_rows, pe_rows[: _PE2 - _SEQ]], axis=0)
    pe2p = jnp.pad(pe2, ((0, 0), (0, _PAD_DIM - _NUM_DIM)))
    out = _embed_sc(x.reshape(-1).astype(jnp.int32), pe2p, table)
    return out[:, :_NUM_DIM].reshape(_BATCH, _SEQ, _NUM_DIM)


# speedup vs baseline: 1.6830x; 1.3290x over previous
import functools

import jax
import jax.numpy as jnp
from jax import lax
from jax.experimental import pallas as pl
from jax.experimental.pallas import tpu as pltpu
from jax.experimental.pallas import tpu_sc as plsc

_NUM_DIM = 64
_PAD_DIM = 128
_BATCH = 4096
_SEQ = 200

_info = plsc.get_sparse_core_info()
_NC, _NS = _info.num_cores, _info.num_subcores
_NW = _NC * _NS
_TOTAL = _BATCH * _SEQ
_ROWS_PER_W = _TOTAL // _NW
_CH = 128
_NCHUNK = _ROWS_PER_W // _CH
_PE2 = _SEQ + _CH - 8
_D = 4
_NG = _NCHUNK // _D


@functools.partial(
    pl.kernel,
    out_type=jax.ShapeDtypeStruct((_TOTAL, _PAD_DIM), jnp.float32),
    mesh=plsc.VectorSubcoreMesh(core_axis_name="c", subcore_axis_name="s"),
    scratch_types=[
        pltpu.VMEM_SHARED((_PE2, _NUM_DIM), jnp.float32),
        [pltpu.VMEM((_CH,), jnp.int32) for _ in range(_D)],
        [pltpu.VMEM((_CH, _NUM_DIM), jnp.float32) for _ in range(_D)],
        [pltpu.SemaphoreType.DMA for _ in range(_D)],
        [pltpu.SemaphoreType.DMA for _ in range(_D)],
        [pltpu.SemaphoreType.DMA for _ in range(_D)],
    ],
    compiler_params=pltpu.CompilerParams(use_tc_tiling_on_sc=False),
)
def _embed_sc(x_hbm, pe_hbm, table_hbm, out_hbm,
              pe_sh, idxs, bufs, sem_i, sem_p, sem_g):
    wid = lax.axis_index("s") * _NC + lax.axis_index("c")
    base = wid * _ROWS_PER_W

    @pl.when(lax.axis_index("s") == 0)
    def _():
        pltpu.sync_copy(pe_hbm, pe_sh)

    plsc.subcore_barrier()

    def fire_inputs(j, c):
        off = base + c * _CH
        r = (c * _CH) % _SEQ
        pltpu.async_copy(x_hbm.at[pl.ds(off, _CH)], idxs[j], sem_i[j])
        pltpu.async_copy(pe_sh.at[pl.ds(r, _CH)], bufs[j], sem_p[j])

    def wait_inputs(j, c):
        off = base + c * _CH
        r = (c * _CH) % _SEQ
        pltpu.make_async_copy(x_hbm.at[pl.ds(off, _CH)], idxs[j],
                              sem_i[j]).wait()
        pltpu.make_async_copy(pe_sh.at[pl.ds(r, _CH)], bufs[j],
                              sem_p[j]).wait()

    for j in range(_D):
        fire_inputs(j, j)

    def group(g, carry):
        for j in range(_D):
            c = g * _D + j
            wait_inputs(j, c)
            pltpu.async_copy(table_hbm.at[idxs[j]], bufs[j], sem_g[j],
                             add=True)
        for j in range(_D):
            c = g * _D + j
            off = base + c * _CH
            pltpu.make_async_copy(table_hbm.at[idxs[j]], bufs[j],
                                  sem_g[j]).wait()
            pltpu.async_copy(bufs[j],
                             out_hbm.at[pl.ds(off, _CH), pl.ds(0, _NUM_DIM)],
                             sem_g[j])
        for j in range(_D):
            c = g * _D + j
            off = base + c * _CH
            pltpu.make_async_copy(
                bufs[j], out_hbm.at[pl.ds(off, _CH), pl.ds(0, _NUM_DIM)],
                sem_g[j]).wait()
            cn = jnp.minimum((g + 1) * _D + j, _NCHUNK - 1)
            fire_inputs(j, cn)
        return carry

    lax.fori_loop(0, _NG, group, 0)

    for j in range(_D):
        wait_inputs(j, _NCHUNK - 1)


def kernel(x, table, pe):
    pe_rows = pe[0, :_SEQ]
    pe2 = jnp.concatenate([pe_rows, pe_rows[: _PE2 - _SEQ]], axis=0)
    out = _embed_sc(x.reshape(-1).astype(jnp.int32), pe2, table)
    return out[:, :_NUM_DIM].reshape(_BATCH, _SEQ, _NUM_DIM)


# speedup vs baseline: 1.6968x; 1.0082x over previous
import functools

import jax
import jax.numpy as jnp
from jax import lax
from jax.experimental import pallas as pl
from jax.experimental.pallas import tpu as pltpu
from jax.experimental.pallas import tpu_sc as plsc

_NUM_DIM = 64
_PAD_DIM = 128
_BATCH = 4096
_SEQ = 200

_info = plsc.get_sparse_core_info()
_NC, _NS = _info.num_cores, _info.num_subcores
_NW = _NC * _NS
_TOTAL = _BATCH * _SEQ
_ROWS_PER_W = _TOTAL // _NW
_CH = 128
_NCHUNK = _ROWS_PER_W // _CH
_PE2 = _SEQ + _CH - 8
_D = 8
_NG = _NCHUNK // _D


@functools.partial(
    pl.kernel,
    out_type=jax.ShapeDtypeStruct((_TOTAL, _PAD_DIM), jnp.float32),
    mesh=plsc.VectorSubcoreMesh(core_axis_name="c", subcore_axis_name="s"),
    scratch_types=[
        pltpu.VMEM_SHARED((_PE2, _NUM_DIM), jnp.float32),
        [pltpu.VMEM((_CH,), jnp.int32) for _ in range(_D)],
        [pltpu.VMEM((_CH, _NUM_DIM), jnp.float32) for _ in range(_D)],
        [pltpu.SemaphoreType.DMA for _ in range(_D)],
        [pltpu.SemaphoreType.DMA for _ in range(_D)],
        [pltpu.SemaphoreType.DMA for _ in range(_D)],
    ],
    compiler_params=pltpu.CompilerParams(use_tc_tiling_on_sc=False),
)
def _embed_sc(x_hbm, pe_hbm, table_hbm, out_hbm,
              pe_sh, idxs, bufs, sem_i, sem_p, sem_g):
    wid = lax.axis_index("s") * _NC + lax.axis_index("c")
    base = wid * _ROWS_PER_W

    @pl.when(lax.axis_index("s") == 0)
    def _():
        pltpu.sync_copy(pe_hbm, pe_sh)

    plsc.subcore_barrier()

    def fire_inputs(j, c):
        off = base + c * _CH
        r = (c * _CH) % _SEQ
        pltpu.async_copy(x_hbm.at[pl.ds(off, _CH)], idxs[j], sem_i[j])
        pltpu.async_copy(pe_sh.at[pl.ds(r, _CH)], bufs[j], sem_p[j])

    def wait_inputs(j, c):
        off = base + c * _CH
        r = (c * _CH) % _SEQ
        pltpu.make_async_copy(x_hbm.at[pl.ds(off, _CH)], idxs[j],
                              sem_i[j]).wait()
        pltpu.make_async_copy(pe_sh.at[pl.ds(r, _CH)], bufs[j],
                              sem_p[j]).wait()

    for j in range(_D):
        fire_inputs(j, j)

    def group(g, carry):
        for j in range(_D):
            c = g * _D + j
            wait_inputs(j, c)
            pltpu.async_copy(table_hbm.at[idxs[j]], bufs[j], sem_g[j],
                             add=True)
        for j in range(_D):
            c = g * _D + j
            off = base + c * _CH
            pltpu.make_async_copy(table_hbm.at[idxs[j]], bufs[j],
                                  sem_g[j]).wait()
            pltpu.async_copy(bufs[j],
                             out_hbm.at[pl.ds(off, _CH), pl.ds(0, _NUM_DIM)],
                             sem_g[j])
        for j in range(_D):
            c = g * _D + j
            off = base + c * _CH
            pltpu.make_async_copy(
                bufs[j], out_hbm.at[pl.ds(off, _CH), pl.ds(0, _NUM_DIM)],
                sem_g[j]).wait()
            cn = jnp.minimum((g + 1) * _D + j, _NCHUNK - 1)
            fire_inputs(j, cn)
        return carry

    lax.fori_loop(0, _NG, group, 0)

    for j in range(_D):
        wait_inputs(j, _NCHUNK - 1)


def kernel(x, table, pe):
    pe_rows = pe[0, :_SEQ]
    pe2 = jnp.concatenate([pe_rows, pe_rows[: _PE2 - _SEQ]], axis=0)
    out = _embed_sc(x.reshape(-1).astype(jnp.int32), pe2, table)
    return out[:, :_NUM_DIM].reshape(_BATCH, _SEQ, _NUM_DIM)
